# baseline (speedup 1.0000x reference)
import dataclasses

import jax
import jax.numpy as jnp
import numpy as np
from jax import lax
from jax.experimental import pallas as pl
from jax.experimental.pallas import tpu as pltpu
from jax.experimental.pallas import tpu_sc as plsc

NC, NS, L = 2, 16, 16
NW = NC * NS
B = 16384
F = 26
D = 16
V_EMB = 2600000
LIN_V = 26000
ROWS_PER_W = B // NW
SUB = 128
NSUB = ROWS_PER_W // SUB
W_IDX_ROWS = F * NSUB
TCB = 4096

_OFFS_FM = np.arange(F, dtype=np.int32) * 100000
_OFFS_LIN = np.arange(F, dtype=np.int32) * 1000


def _fm_body(fm_idx_h, lin_idx_h, emb_h, wfl_h, wl_h, bias_h,
             out_h, idx_e, idx_l, rows_v, fval_v, lin_tab, bias_v,
             out_v, sem):
    wid = lax.axis_index("c") * NS + lax.axis_index("s")
    pltpu.sync_copy(bias_h, bias_v)
    pltpu.sync_copy(wl_h, lin_tab)
    pltpu.sync_copy(fm_idx_h.at[pl.ds(wid * W_IDX_ROWS, W_IDX_ROWS)], idx_e)
    pltpu.sync_copy(lin_idx_h.at[pl.ds(wid * W_IDX_ROWS * 128,
                                       W_IDX_ROWS * 128)], idx_l)

    zeros16 = jnp.zeros((16,), jnp.float32)

    @pl.loop(0, ROWS_PER_W // 16)
    def _(i):
        out_v[pl.ds(i * 16, 16)] = zeros16

    @pl.loop(0, NSUB)
    def _(c):
        copies = []
        for f in range(F):
            copies.append(pltpu.async_copy(
                emb_h.at[idx_e.at[f * NSUB + c]],
                rows_v.at[pl.ds(f * 128, 128)], sem))
        for f in range(F):
            copies.append(pltpu.async_copy(
                wfl_h.at[idx_e.at[f * NSUB + c]],
                fval_v.at[pl.ds(f * 128, 128)], sem))
        for cp in copies:
            cp.wait()

        @pl.loop(0, SUB)
        def _(r):
            s = jnp.zeros((16,), jnp.float32)
            q = jnp.zeros((16,), jnp.float32)
            for f in range(F):
                v = rows_v[f * 128 + r]
                s = s + v
                q = q + v * v
            red = 0.5 * (s * s - q)
            pos = jnp.full((16,), c * SUB + r, jnp.int32)
            plsc.addupdate_scatter(out_v, [pos], red)

        @pl.loop(0, SUB // 16)
        def _(k):
            acc = jnp.zeros((16,), jnp.float32)
            for f in range(F):
                acc = acc + fval_v[pl.ds(f * 128 + k * 16, 16)]
            for f in range(F):
                lidx = idx_l[pl.ds((f * NSUB + c) * 128 + k * 16, 16)]
                acc = acc + plsc.load_gather(lin_tab, [lidx])
            o = pl.ds(c * SUB + k * 16, 16)
            out_v[o] = out_v[o] + acc

    bias_vec = bias_v[...]

    @pl.loop(0, ROWS_PER_W // 16)
    def _(i):
        zv = out_v[pl.ds(i * 16, 16)] + bias_vec
        out_v[pl.ds(i * 16, 16)] = 1.0 / (1.0 + jnp.exp(-zv))

    pltpu.sync_copy(out_v, out_h.at[pl.ds(wid * ROWS_PER_W, ROWS_PER_W)])


@jax.jit
def _fm_model(fm_idx_g, lin_idx_g, w_emb_rows, wfl, wl, bias16):
    mesh = plsc.VectorSubcoreMesh(core_axis_name="c", subcore_axis_name="s")
    cp = pltpu.CompilerParams()
    for fld, val in (("needs_layout_passes", False),
                     ("use_tc_tiling_on_sc", False)):
        if fld in pltpu.CompilerParams.__dataclass_fields__:
            cp = dataclasses.replace(cp, **{fld: val})
    krn = pl.kernel(
        _fm_body,
        out_type=jax.ShapeDtypeStruct((B,), jnp.float32),
        mesh=mesh,
        compiler_params=cp,
        scratch_types=[
            pltpu.VMEM((W_IDX_ROWS, 128), jnp.int32),
            pltpu.VMEM((W_IDX_ROWS * 128,), jnp.int32),
            pltpu.VMEM((SUB * F, D), jnp.float32),
            pltpu.VMEM((SUB * F,), jnp.float32),
            pltpu.VMEM((LIN_V,), jnp.float32),
            pltpu.VMEM((16,), jnp.float32),
            pltpu.VMEM((ROWS_PER_W,), jnp.float32),
            pltpu.SemaphoreType.DMA,
        ],
    )
    return krn(fm_idx_g, lin_idx_g, w_emb_rows, wfl, wl, bias16)


def _worker_major(idx_t):
    return (idx_t.reshape(F, NW, NSUB, 128)
            .swapaxes(0, 1)
            .reshape(NW * W_IDX_ROWS, 128))


def kernel(fm_x, linear_x, W_embed, W_fm_linear, b_fm, W_lin, b_lin):
    fm_idx_t = fm_x.T.astype(jnp.int32) + jnp.asarray(_OFFS_FM)[:, None]
    lin_idx_t = linear_x.T.astype(jnp.int32) + jnp.asarray(_OFFS_LIN)[:, None]

    fm_idx_g = _worker_major(fm_idx_t)
    lin_idx_g = _worker_major(lin_idx_t).reshape(-1)

    w128 = lax.optimization_barrier(
        jnp.transpose(jnp.transpose(W_embed).reshape(D, V_EMB // D, D),
                      (1, 2, 0)).reshape(V_EMB * D // 128, 128))
    w_emb_rows = w128.reshape(V_EMB, D)
    bias16 = jnp.broadcast_to((b_fm + b_lin).astype(jnp.float32), (16,))
    return _fm_model(fm_idx_g, lin_idx_g, w_emb_rows,
                     W_fm_linear[:, 0], W_lin[:, 0], bias16)


# speedup vs baseline: 12.0075x; 12.0075x over previous
import dataclasses

import jax
import jax.numpy as jnp
import numpy as np
from jax import lax
from jax.experimental import pallas as pl
from jax.experimental.pallas import tpu as pltpu
from jax.experimental.pallas import tpu_sc as plsc

NC, NS, L = 2, 16, 16
NW = NC * NS
B = 16384
F = 26
D = 16
V_EMB = 2600000
LIN_V = 26000
ROWS_PER_W = B // NW
SUB = 128
NSUB = ROWS_PER_W // SUB
W_IDX_ROWS = F * NSUB
TCB = 4096

_OFFS_FM = np.arange(F, dtype=np.int32) * 100000
_OFFS_LIN = np.arange(F, dtype=np.int32) * 1000


def _fm_body(fm_idx_h, lin_idx_h, emb_h, wfl_h, wl_h, bias_h,
             out_h, idx_e, idx_l, rows_v, fval_v, lin_tab, bias_v,
             out_v, sem):
    wid = lax.axis_index("c") * NS + lax.axis_index("s")
    pltpu.sync_copy(bias_h, bias_v)
    pltpu.sync_copy(wl_h, lin_tab)
    pltpu.sync_copy(fm_idx_h.at[pl.ds(wid * W_IDX_ROWS, W_IDX_ROWS)], idx_e)
    pltpu.sync_copy(lin_idx_h.at[pl.ds(wid * W_IDX_ROWS * 128,
                                       W_IDX_ROWS * 128)], idx_l)

    zeros16 = jnp.zeros((16,), jnp.float32)

    @pl.loop(0, ROWS_PER_W // 16)
    def _(i):
        out_v[pl.ds(i * 16, 16)] = zeros16

    @pl.loop(0, NSUB)
    def _(c):
        copies = []
        for f in range(F):
            copies.append(pltpu.async_copy(
                emb_h.at[idx_e.at[f * NSUB + c]],
                rows_v.at[pl.ds(f * 128, 128)], sem))
        for f in range(F):
            copies.append(pltpu.async_copy(
                wfl_h.at[idx_e.at[f * NSUB + c]],
                fval_v.at[pl.ds(f * 128, 128)], sem))
        for cp in copies:
            cp.wait()

        @pl.loop(0, SUB)
        def _(r):
            s = jnp.zeros((16,), jnp.float32)
            q = jnp.zeros((16,), jnp.float32)
            for f in range(F):
                v = rows_v[f * 128 + r]
                s = s + v
                q = q + v * v
            red = 0.5 * (s * s - q)
            pos = jnp.full((16,), c * SUB + r, jnp.int32)
            plsc.addupdate_scatter(out_v, [pos], red)

        @pl.loop(0, SUB // 16)
        def _(k):
            acc = jnp.zeros((16,), jnp.float32)
            for f in range(F):
                acc = acc + fval_v[pl.ds(f * 128 + k * 16, 16)]
            for f in range(F):
                lidx = idx_l[pl.ds((f * NSUB + c) * 128 + k * 16, 16)]
                acc = acc + plsc.load_gather(lin_tab, [lidx])
            o = pl.ds(c * SUB + k * 16, 16)
            out_v[o] = out_v[o] + acc

    bias_vec = bias_v[...]

    @pl.loop(0, ROWS_PER_W // 16)
    def _(i):
        zv = out_v[pl.ds(i * 16, 16)] + bias_vec
        out_v[pl.ds(i * 16, 16)] = 1.0 / (1.0 + jnp.exp(-zv))

    pltpu.sync_copy(out_v, out_h.at[pl.ds(wid * ROWS_PER_W, ROWS_PER_W)])


@jax.jit
def _fm_model(fm_idx_g, lin_idx_g, w_emb_rows, wfl, wl, bias16):
    mesh = plsc.VectorSubcoreMesh(core_axis_name="c", subcore_axis_name="s")
    cp = pltpu.CompilerParams()
    for fld, val in (("needs_layout_passes", False),
                     ("use_tc_tiling_on_sc", False)):
        if fld in pltpu.CompilerParams.__dataclass_fields__:
            cp = dataclasses.replace(cp, **{fld: val})
    krn = pl.kernel(
        _fm_body,
        out_type=jax.ShapeDtypeStruct((B,), jnp.float32),
        mesh=mesh,
        compiler_params=cp,
        scratch_types=[
            pltpu.VMEM((W_IDX_ROWS, 128), jnp.int32),
            pltpu.VMEM((W_IDX_ROWS * 128,), jnp.int32),
            pltpu.VMEM((SUB * F, D), jnp.float32),
            pltpu.VMEM((SUB * F,), jnp.float32),
            pltpu.VMEM((LIN_V,), jnp.float32),
            pltpu.VMEM((16,), jnp.float32),
            pltpu.VMEM((ROWS_PER_W,), jnp.float32),
            pltpu.SemaphoreType.DMA,
        ],
    )
    return krn(fm_idx_g, lin_idx_g, w_emb_rows, wfl, wl, bias16)


def _worker_major(idx_t):
    return (idx_t.reshape(F, NW, NSUB, 128)
            .swapaxes(0, 1)
            .reshape(NW * W_IDX_ROWS, 128))


def kernel(fm_x, linear_x, W_embed, W_fm_linear, b_fm, W_lin, b_lin):
    fm_idx_t = fm_x.T.astype(jnp.int32) + jnp.asarray(_OFFS_FM)[:, None]
    lin_idx_t = linear_x.T.astype(jnp.int32) + jnp.asarray(_OFFS_LIN)[:, None]

    fm_idx_g = _worker_major(fm_idx_t)
    lin_idx_g = _worker_major(lin_idx_t).reshape(-1)

    w_emb_rows = W_embed
    bias16 = jnp.broadcast_to((b_fm + b_lin).astype(jnp.float32), (16,))
    return _fm_model(fm_idx_g, lin_idx_g, w_emb_rows,
                     W_fm_linear[:, 0], W_lin[:, 0], bias16)
